# Initial kernel scaffold; baseline (speedup 1.0000x reference)
import jax
import jax.numpy as jnp
from jax.experimental import pallas as pl


def kernel(x, edge_index, W_dist, b_dist, W_self, W_neigh, b_sage, W_lin, b_lin):
    raise NotImplementedError("write your pallas kernel here")


# speedup vs baseline: 11.4104x; 11.4104x over previous
import functools
import jax
import jax.numpy as jnp
from jax import lax
from jax.experimental import pallas as pl
from jax.experimental.pallas import tpu as pltpu
from jax.experimental.pallas import tpu_sc as plsc

N = 10000
E = 320000
D = 128
H = 64
C = 2

NPAD = 10240
NH = 5120
NT = 320
EC = E // 16
NSUB = 5
ECH = EC // NSUB
VECS = ECH // 16
ROWS = 32
LO = 19
NB = 128
WINDOWS = [(44, 7), (37, 7), (30, 7), (23, 7), (16, 7), (9, 7), (2, 7), (0, 2)]
HHALF = (NT // 2) * NB

_mesh = plsc.VectorSubcoreMesh(core_axis_name="c", subcore_axis_name="s",
                               num_cores=2, num_subcores=16)
_sc_params = pltpu.CompilerParams(needs_layout_passes=False)


def _det_masks(p):
    mh = ml = 0
    for s0, w in WINDOWS[:p]:
        m = (1 << w) - 1
        if s0 >= LO:
            mh |= m << (s0 - LO)
        elif s0 + w <= LO:
            ml |= m << s0
        else:
            mh |= m >> (LO - s0)
            ml |= (m & ((1 << (LO - s0)) - 1)) << s0
    return mh, ml


def _bucket(s0, w, bts, loidx):
    m = (1 << w) - 1
    if s0 >= LO:
        return (bts >> (s0 - LO)) & m
    if s0 + w <= LO:
        return (loidx >> s0) & m
    return ((bts << (LO - s0)) | (loidx >> s0)) & m


def _tc_pre_body(x_ref, wd_ref, bd_ref, ws_ref, wn_ref, bs_ref,
                 s_ref, score_ref, z_ref, y_ref):
    x = x_ref[...]
    s = jnp.dot(x, wd_ref[...], preferred_element_type=jnp.float32) + bd_ref[...]
    s_ref[...] = s
    score_ref[...] = jnp.tanh(s[:, 0:1])
    z_ref[...] = jnp.dot(x, ws_ref[...], preferred_element_type=jnp.float32) + bs_ref[...]
    y_ref[...] = jnp.dot(x, wn_ref[...], preferred_element_type=jnp.float32)


def _tc_pre(x, W_dist, b_dist, W_self, W_neigh, b_sage):
    return pl.pallas_call(
        _tc_pre_body,
        out_shape=[
            jax.ShapeDtypeStruct((N, 2), jnp.float32),
            jax.ShapeDtypeStruct((N, 1), jnp.float32),
            jax.ShapeDtypeStruct((N, H), jnp.float32),
            jax.ShapeDtypeStruct((N, H), jnp.float32),
        ],
    )(x, W_dist, b_dist.reshape(1, 2), W_self, W_neigh, b_sage.reshape(1, H))


EB = E // 32


def _sc_bits_body(score_hbm, src_hbm, dst_hbm, bits_hbm,
                  score_v, src_v, dst_v, out_v):
    c = lax.axis_index("c")
    t = lax.axis_index("s")
    w = t * 2 + c
    base = w * EB
    pltpu.sync_copy(score_hbm, score_v)
    pltpu.sync_copy(src_hbm.at[pl.ds(base, EB)], src_v)
    pltpu.sync_copy(dst_hbm.at[pl.ds(base, EB)], dst_v)

    def body(i, carry):
        sl = pl.ds(i * 16, 16)
        a = plsc.load_gather(score_v, [src_v[sl]])
        b = plsc.load_gather(score_v, [dst_v[sl]])
        out_v[sl] = plsc.bitcast(jnp.abs(a - b), jnp.int32)
        return carry

    lax.fori_loop(0, EB // 16, body, 0)
    pltpu.sync_copy(out_v, bits_hbm.at[pl.ds(base, EB)])


_sc_bits = functools.partial(
    pl.kernel,
    out_type=jax.ShapeDtypeStruct((E,), jnp.int32),
    mesh=_mesh,
    compiler_params=_sc_params,
    scratch_types=[
        pltpu.VMEM((N,), jnp.float32),
        pltpu.VMEM((EB,), jnp.int32),
        pltpu.VMEM((EB,), jnp.int32),
        pltpu.VMEM((EB,), jnp.int32),
    ],
)(_sc_bits_body)


def _i32c(v):
    return v - (1 << 32) if v >= (1 << 31) else v


def _make_pass(p):
    s0, w = WINDOWS[p]
    mh, ml = _det_masks(p)
    mh, ml = _i32c(mh), _i32c(ml)
    first = p == 0

    def body(*refs):
        zero16 = jnp.zeros((16,), jnp.int32)
        if first:
            (dst_hbm, bits_hbm,
             ph_out, pl_out, r_out, cnt_out,
             ph_v, pl_v, dst_v, bits_v, idx_st, val_st,
             hist_v, r_v, pho_v, plo_v, ro_v, cnto_v, hist_sh) = refs
        else:
            (dst_hbm, bits_hbm, ph_hbm, plh_hbm, r_hbm,
             ph_out, pl_out, r_out,
             ph_v, pl_v, dst_v, bits_v, idx_st, val_st,
             hist_v, r_v, pho_v, plo_v, ro_v, cnto_v, hist_sh) = refs
        c = lax.axis_index("c")
        t = lax.axis_index("s")
        iota = lax.iota(jnp.int32, 16)

        def zb(i, carry):
            hist_v[pl.ds(i * 16, 16)] = zero16
            return carry
        lax.fori_loop(0, HHALF // 16, zb, 0)
        pltpu.sync_copy(hist_v, hist_sh.at[pl.ds(t * NT * NB, HHALF)])
        pltpu.sync_copy(hist_v, hist_sh.at[pl.ds(t * NT * NB + HHALF, HHALF)])
        if not first:
            pltpu.sync_copy(ph_hbm, ph_v)
            pltpu.sync_copy(plh_hbm, pl_v)
        plsc.subcore_barrier()

        for sub in range(NSUB):
            base = t * EC + sub * ECH
            pltpu.sync_copy(dst_hbm.at[pl.ds(base, ECH)], dst_v)
            pltpu.sync_copy(bits_hbm.at[pl.ds(base, ECH)], bits_v)

            for u in range(VECS % 8, 8):
                val_st[VECS // 8, pl.ds(u * 16, 16)] = zero16
                idx_st[VECS // 8, pl.ds(u * 16, 16)] = zero16
            for rr in range(VECS // 8 + 1, ROWS):
                for u in range(8):
                    val_st[rr, pl.ds(u * 16, 16)] = zero16
                    idx_st[rr, pl.ds(u * 16, 16)] = zero16

            def eb(k, carry):
                sl = pl.ds(k * 16, 16)
                d = dst_v[sl]
                bts = bits_v[sl]
                loidx = base + k * 16 + iota
                if first:
                    act = jnp.full((16,), True)
                else:
                    phd = plsc.load_gather(ph_v, [d])
                    pld = plsc.load_gather(pl_v, [d])
                    act = ((bts & mh) == phd) & ((loidx & ml) == pld)
                b = _bucket(s0, w, bts, loidx)
                inhalf = (d >= c * NH) & (d < (c + 1) * NH)
                val = (act & inhalf).astype(jnp.int32)
                dl = jnp.clip(d - c * NH, 0, NH - 1)
                slot = dl * NB + b
                row = k // 8
                col = (k % 8) * 16
                idx_st[row, pl.ds(col, 16)] = slot
                val_st[row, pl.ds(col, 16)] = val
                return carry
            lax.fori_loop(0, VECS, eb, 0)

            def fire(j, carry):
                pltpu.sync_copy(val_st.at[j], hist_sh.at[idx_st.at[j]], add=True)
                return carry
            lax.fori_loop(0, ROWS, fire, 0)
        plsc.subcore_barrier()

        nb0 = c * NH + t * NT
        if not first:
            pltpu.sync_copy(r_hbm.at[pl.ds(nb0, NT)], r_v)

        for h in range(2):
            pltpu.sync_copy(
                hist_sh.at[pl.ds(t * NT * NB + h * HHALF, HHALF)], hist_v)

            def ng(g, carry, h=h):
                tn = h * (NT // 2) + g * 16
                gb = (g * 16 + iota) * NB
                if first:
                    def sb(b, cum):
                        return cum + plsc.load_gather(hist_v, [gb + b])
                    deg = lax.fori_loop(0, 1 << w, sb, zero16)
                    keep = jnp.maximum(deg >> 1, 1)
                    r = keep
                    cnto_v[pl.ds(tn, 16)] = jnp.where(deg > 0, keep, 0)
                    ph = zero16
                    plv = zero16
                else:
                    r = r_v[pl.ds(tn, 16)]
                    ph = ph_v[pl.ds(nb0 + tn, 16)]
                    plv = pl_v[pl.ds(nb0 + tn, 16)]

                def scan(b, carry):
                    cum, fnd, bsel, cumb = carry
                    cb = plsc.load_gather(hist_v, [gb + b])
                    cum2 = cum + cb
                    hit = (~fnd) & (cum2 >= r)
                    bsel = jnp.where(hit, b, bsel)
                    cumb = jnp.where(hit, cum, cumb)
                    return cum2, fnd | hit, bsel, cumb

                cum, fnd, bsel, cumb = lax.fori_loop(
                    0, 1 << w, scan,
                    (zero16, jnp.full((16,), False), zero16, zero16))
                rnew = jnp.where(fnd, r - cumb, r)
                if s0 >= LO:
                    ph2 = jnp.where(fnd, ph | (bsel << (s0 - LO)), ph)
                    pl2 = plv
                elif s0 + w <= LO:
                    ph2 = ph
                    pl2 = jnp.where(fnd, plv | (bsel << s0), plv)
                else:
                    ph2 = jnp.where(fnd, ph | (bsel >> (LO - s0)), ph)
                    pl2 = jnp.where(
                        fnd, plv | ((bsel & ((1 << (LO - s0)) - 1)) << s0), plv)
                sl = pl.ds(tn, 16)
                pho_v[sl] = ph2
                plo_v[sl] = pl2
                ro_v[sl] = rnew
                return carry
            lax.fori_loop(0, NT // 32, ng, 0)

        osl = pl.ds(nb0, NT)
        pltpu.sync_copy(pho_v, ph_out.at[osl])
        pltpu.sync_copy(plo_v, pl_out.at[osl])
        pltpu.sync_copy(ro_v, r_out.at[osl])
        if first:
            pltpu.sync_copy(cnto_v, cnt_out.at[osl])

    i32v = jax.ShapeDtypeStruct((NPAD,), jnp.int32)
    out_type = [i32v, i32v, i32v, i32v] if first else [i32v, i32v, i32v]
    return functools.partial(
        pl.kernel,
        out_type=out_type,
        mesh=_mesh,
        compiler_params=_sc_params,
        scratch_types=[
            pltpu.VMEM((NPAD,), jnp.int32),
            pltpu.VMEM((NPAD,), jnp.int32),
            pltpu.VMEM((ECH,), jnp.int32),
            pltpu.VMEM((ECH,), jnp.int32),
            pltpu.VMEM((ROWS, 128), jnp.int32),
            pltpu.VMEM((ROWS, 128), jnp.int32),
            pltpu.VMEM((HHALF,), jnp.int32),
            pltpu.VMEM((NT,), jnp.int32),
            pltpu.VMEM((NT,), jnp.int32),
            pltpu.VMEM((NT,), jnp.int32),
            pltpu.VMEM((NT,), jnp.int32),
            pltpu.VMEM((NT,), jnp.int32),
            pltpu.VMEM_SHARED((NH * NB,), jnp.int32),
        ],
    )(body)


_passes = [_make_pass(p) for p in range(len(WINDOWS))]


NHA = NH + 8


def _sc_agg_body(src_hbm, dst_hbm, bits_hbm, ph_hbm, plh_hbm, y_hbm, agg_out,
                 ph_v, pl_v, src_v, dst_v, bits_v, ssel_v, dsel_v,
                 rows_v, idx2d, zbuf, agg_sh, sem):
    c = lax.axis_index("c")
    t = lax.axis_index("s")
    iota = lax.iota(jnp.int32, 16)
    zf16 = jnp.zeros((16,), jnp.float32)

    def zb(i, carry):
        zbuf[i // 4, pl.ds((i % 4) * 16, 16)] = zf16
        return carry
    lax.fori_loop(0, 32 * 4, zb, 0)

    def za(i, carry):
        pltpu.sync_copy(zbuf, agg_sh.at[pl.ds(t * NT + i * 32, 32)])
        return carry
    lax.fori_loop(0, NT // 32, za, 0)

    @pl.when(t == 15)
    def _():
        pltpu.sync_copy(zbuf.at[pl.ds(0, 8)], agg_sh.at[pl.ds(NH, 8)])

    pltpu.sync_copy(ph_hbm, ph_v)
    pltpu.sync_copy(plh_hbm, pl_v)
    plsc.subcore_barrier()

    nsel = jnp.int32(0)
    for sub in range(NSUB):
        base = t * EC + sub * ECH
        pltpu.sync_copy(src_hbm.at[pl.ds(base, ECH)], src_v)
        pltpu.sync_copy(dst_hbm.at[pl.ds(base, ECH)], dst_v)
        pltpu.sync_copy(bits_hbm.at[pl.ds(base, ECH)], bits_v)

        def eb(k, ns):
            sl = pl.ds(k * 16, 16)
            d = dst_v[sl]
            sr = src_v[sl]
            bts = bits_v[sl]
            loidx = base + k * 16 + iota
            phd = plsc.load_gather(ph_v, [d])
            pld = plsc.load_gather(pl_v, [d])
            kept = (bts < phd) | ((bts == phd) & (loidx <= pld))
            m = kept & (d >= c * NH) & (d < (c + 1) * NH)
            plsc.store_compressed(ssel_v.at[pl.ds(ns, 16)], sr, mask=m)
            plsc.store_compressed(dsel_v.at[pl.ds(ns, 16)], d - c * NH, mask=m)
            return ns + jnp.sum(m.astype(jnp.int32))
        nsel = lax.fori_loop(0, VECS, eb, nsel)

    dummy_s = jnp.zeros((16,), jnp.int32)
    dummy_d = jnp.full((16,), NH, jnp.int32)

    def pad(i, carry):
        ssel_v[pl.ds(nsel + i * 16, 16)] = dummy_s
        dsel_v[pl.ds(nsel + i * 16, 16)] = dummy_d
        return carry
    lax.fori_loop(0, 8, pad, 0)

    nblk = (nsel + 127) // 128

    def blk(j, carry):
        pltpu.async_copy(
            y_hbm.at[ssel_v.at[pl.ds(j * 128, 128)]], rows_v, sem).wait()
        for u in range(8):
            idx2d[0, pl.ds(u * 16, 16)] = dsel_v[pl.ds(j * 128 + u * 16, 16)]
        pltpu.sync_copy(rows_v, agg_sh.at[idx2d.at[0]], add=True)
        return carry
    lax.fori_loop(0, nblk, blk, 0)
    plsc.subcore_barrier()

    pltpu.sync_copy(agg_sh.at[pl.ds(t * NT, NT)],
                    agg_out.at[c, pl.ds(t * NT, NT)])


_sc_agg = functools.partial(
        pl.kernel,
        out_type=jax.ShapeDtypeStruct((2, NH, H), jnp.float32),
        mesh=_mesh,
        compiler_params=pltpu.CompilerParams(
            needs_layout_passes=False, use_tc_tiling_on_sc=False),
        scratch_types=[
            pltpu.VMEM((NPAD,), jnp.int32),
            pltpu.VMEM((NPAD,), jnp.int32),
            pltpu.VMEM((ECH,), jnp.int32),
            pltpu.VMEM((ECH,), jnp.int32),
            pltpu.VMEM((ECH,), jnp.int32),
            pltpu.VMEM((EC + 512,), jnp.int32),
            pltpu.VMEM((EC + 512,), jnp.int32),
            pltpu.VMEM((128, H), jnp.float32),
            pltpu.VMEM((1, 128), jnp.int32),
            pltpu.VMEM((32, H), jnp.float32),
            pltpu.VMEM_SHARED((NHA, H), jnp.float32),
            pltpu.SemaphoreType.DMA,
        ],
    )(_sc_agg_body)


def _tc_post_body(z_ref, agg_ref, cnt_ref, wl_ref, bl_ref, out_ref):
    agg = jnp.concatenate([agg_ref[0], agg_ref[1]], axis=0)[:N]
    cnt = cnt_ref[...][:N].astype(jnp.float32)
    mean = agg / jnp.maximum(cnt, 1.0)
    h = jax.nn.relu(z_ref[...] + mean)
    out_ref[...] = jnp.dot(h, wl_ref[...], preferred_element_type=jnp.float32) + bl_ref[...]


def _tc_post(z, agg, cnt, W_lin, b_lin):
    return pl.pallas_call(
        _tc_post_body,
        out_shape=jax.ShapeDtypeStruct((N, C), jnp.float32),
    )(z, agg, cnt.reshape(NPAD, 1), W_lin, b_lin.reshape(1, C))


def kernel(x, edge_index, W_dist, b_dist, W_self, W_neigh, b_sage, W_lin, b_lin):
    src = edge_index[0]
    dst = edge_index[1]
    s, score, z, y = _tc_pre(x, W_dist, b_dist, W_self, W_neigh, b_sage)
    score1 = score.reshape(N)
    bits = _sc_bits(score1, src, dst)
    ph, plv, r, cnt = _passes[0](dst, bits)
    for p in range(1, len(WINDOWS)):
        ph, plv, r = _passes[p](dst, bits, ph, plv, r)
    agg = _sc_agg(src, dst, bits, ph, plv, y)
    logits = _tc_post(z, agg, cnt, W_lin, b_lin)
    return logits, s


# speedup vs baseline: 11.9029x; 1.0432x over previous
import functools
import jax
import jax.numpy as jnp
from jax import lax
from jax.experimental import pallas as pl
from jax.experimental.pallas import tpu as pltpu
from jax.experimental.pallas import tpu_sc as plsc

N = 10000
E = 320000
D = 128
H = 64
C = 2

NPAD = 10240
NH = 5120
NT = 320
EC = E // 16
NSUB = 5
ECH = EC // NSUB
VECS = ECH // 16
ROWS = 32
LO = 19
NB = 128
WINDOWS = [(44, 7), (37, 7), (30, 7), (23, 7), (16, 7), (9, 7), (2, 7), (0, 2)]
HHALF = (NT // 2) * NB

_mesh = plsc.VectorSubcoreMesh(core_axis_name="c", subcore_axis_name="s",
                               num_cores=2, num_subcores=16)
_sc_params = pltpu.CompilerParams(needs_layout_passes=False)


def _det_masks(p):
    mh = ml = 0
    for s0, w in WINDOWS[:p]:
        m = (1 << w) - 1
        if s0 >= LO:
            mh |= m << (s0 - LO)
        elif s0 + w <= LO:
            ml |= m << s0
        else:
            mh |= m >> (LO - s0)
            ml |= (m & ((1 << (LO - s0)) - 1)) << s0
    return mh, ml


def _bucket(s0, w, bts, loidx):
    m = (1 << w) - 1
    if s0 >= LO:
        return (bts >> (s0 - LO)) & m
    if s0 + w <= LO:
        return (loidx >> s0) & m
    return ((bts << (LO - s0)) | (loidx >> s0)) & m


def _tc_pre_body(x_ref, wd_ref, bd_ref, ws_ref, wn_ref, bs_ref,
                 s_ref, score_ref, z_ref, y_ref):
    x = x_ref[...]
    s = jnp.dot(x, wd_ref[...], preferred_element_type=jnp.float32) + bd_ref[...]
    s_ref[...] = s
    score_ref[...] = jnp.tanh(s[:, 0:1])
    z_ref[...] = jnp.dot(x, ws_ref[...], preferred_element_type=jnp.float32) + bs_ref[...]
    y_ref[...] = jnp.dot(x, wn_ref[...], preferred_element_type=jnp.float32)


def _tc_pre(x, W_dist, b_dist, W_self, W_neigh, b_sage):
    return pl.pallas_call(
        _tc_pre_body,
        out_shape=[
            jax.ShapeDtypeStruct((N, 2), jnp.float32),
            jax.ShapeDtypeStruct((N, 1), jnp.float32),
            jax.ShapeDtypeStruct((N, H), jnp.float32),
            jax.ShapeDtypeStruct((N, H), jnp.float32),
        ],
    )(x, W_dist, b_dist.reshape(1, 2), W_self, W_neigh, b_sage.reshape(1, H))


EB = E // 32


def _sc_bits_body(score_hbm, src_hbm, dst_hbm, bits_hbm,
                  score_v, src_v, dst_v, out_v):
    c = lax.axis_index("c")
    t = lax.axis_index("s")
    w = t * 2 + c
    base = w * EB
    pltpu.sync_copy(score_hbm, score_v)
    pltpu.sync_copy(src_hbm.at[pl.ds(base, EB)], src_v)
    pltpu.sync_copy(dst_hbm.at[pl.ds(base, EB)], dst_v)

    def body(i, carry):
        sl = pl.ds(i * 16, 16)
        a = plsc.load_gather(score_v, [src_v[sl]])
        b = plsc.load_gather(score_v, [dst_v[sl]])
        out_v[sl] = plsc.bitcast(jnp.abs(a - b), jnp.int32)
        return carry

    lax.fori_loop(0, EB // 16, body, 0)
    pltpu.sync_copy(out_v, bits_hbm.at[pl.ds(base, EB)])


_sc_bits = functools.partial(
    pl.kernel,
    out_type=jax.ShapeDtypeStruct((E,), jnp.int32),
    mesh=_mesh,
    compiler_params=_sc_params,
    scratch_types=[
        pltpu.VMEM((N,), jnp.float32),
        pltpu.VMEM((EB,), jnp.int32),
        pltpu.VMEM((EB,), jnp.int32),
        pltpu.VMEM((EB,), jnp.int32),
    ],
)(_sc_bits_body)


def _i32c(v):
    return v - (1 << 32) if v >= (1 << 31) else v


def _make_pass(p):
    s0, w = WINDOWS[p]
    mh, ml = _det_masks(p)
    mh, ml = _i32c(mh), _i32c(ml)
    first = p == 0

    def body(*refs):
        zero16 = jnp.zeros((16,), jnp.int32)
        if first:
            (dst_hbm, bits_hbm,
             ph_out, pl_out, r_out, cnt_out,
             ph_v, pl_v, dst_v, bits_v, idx_st, val_st,
             hist_v, r_v, pho_v, plo_v, ro_v, cnto_v, hist_sh, sem) = refs
        else:
            (dst_hbm, bits_hbm, ph_hbm, plh_hbm, r_hbm,
             ph_out, pl_out, r_out,
             ph_v, pl_v, dst_v, bits_v, idx_st, val_st,
             hist_v, r_v, pho_v, plo_v, ro_v, cnto_v, hist_sh, sem) = refs
        c = lax.axis_index("c")
        t = lax.axis_index("s")
        iota = lax.iota(jnp.int32, 16)

        def zb(i, carry):
            hist_v[pl.ds(i * 16, 16)] = zero16
            return carry
        lax.fori_loop(0, HHALF // 16, zb, 0)
        pltpu.sync_copy(hist_v, hist_sh.at[pl.ds(t * NT * NB, HHALF)])
        pltpu.sync_copy(hist_v, hist_sh.at[pl.ds(t * NT * NB + HHALF, HHALF)])
        if not first:
            pltpu.sync_copy(ph_hbm, ph_v)
            pltpu.sync_copy(plh_hbm, pl_v)
        plsc.subcore_barrier()

        for sub in range(NSUB):
            base = t * EC + sub * ECH
            cp1 = pltpu.async_copy(dst_hbm.at[pl.ds(base, ECH)], dst_v, sem)
            cp2 = pltpu.async_copy(bits_hbm.at[pl.ds(base, ECH)], bits_v, sem)
            cp1.wait()
            cp2.wait()

            for u in range(VECS % 8, 8):
                val_st[VECS // 8, pl.ds(u * 16, 16)] = zero16
                idx_st[VECS // 8, pl.ds(u * 16, 16)] = zero16
            for rr in range(VECS // 8 + 1, ROWS):
                for u in range(8):
                    val_st[rr, pl.ds(u * 16, 16)] = zero16
                    idx_st[rr, pl.ds(u * 16, 16)] = zero16

            def eb(k, carry):
                sl = pl.ds(k * 16, 16)
                d = dst_v[sl]
                bts = bits_v[sl]
                loidx = base + k * 16 + iota
                if first:
                    act = jnp.full((16,), True)
                else:
                    phd = plsc.load_gather(ph_v, [d])
                    pld = plsc.load_gather(pl_v, [d])
                    act = ((bts & mh) == phd) & ((loidx & ml) == pld)
                b = _bucket(s0, w, bts, loidx)
                inhalf = (d >= c * NH) & (d < (c + 1) * NH)
                val = (act & inhalf).astype(jnp.int32)
                dl = jnp.clip(d - c * NH, 0, NH - 1)
                slot = dl * NB + b
                row = k // 8
                col = (k % 8) * 16
                idx_st[row, pl.ds(col, 16)] = slot
                val_st[row, pl.ds(col, 16)] = val
                return carry
            lax.fori_loop(0, VECS, eb, 0)

            def fire(j, carry):
                pltpu.async_copy(
                    val_st.at[j], hist_sh.at[idx_st.at[j]], sem, add=True)
                return carry
            lax.fori_loop(0, ROWS, fire, 0)

            def drain(j, carry):
                pltpu.make_async_copy(
                    val_st.at[j], hist_sh.at[idx_st.at[j]], sem).wait()
                return carry
            lax.fori_loop(0, ROWS, drain, 0)
        plsc.subcore_barrier()

        nb0 = c * NH + t * NT
        if not first:
            pltpu.sync_copy(r_hbm.at[pl.ds(nb0, NT)], r_v)

        for h in range(2):
            pltpu.sync_copy(
                hist_sh.at[pl.ds(t * NT * NB + h * HHALF, HHALF)], hist_v)

            def ng(g, carry, h=h):
                tn = h * (NT // 2) + g * 16
                gb = (g * 16 + iota) * NB
                if first:
                    def sb(b, cum):
                        return cum + plsc.load_gather(hist_v, [gb + b])
                    deg = lax.fori_loop(0, 1 << w, sb, zero16)
                    keep = jnp.maximum(deg >> 1, 1)
                    r = keep
                    cnto_v[pl.ds(tn, 16)] = jnp.where(deg > 0, keep, 0)
                    ph = zero16
                    plv = zero16
                else:
                    r = r_v[pl.ds(tn, 16)]
                    ph = ph_v[pl.ds(nb0 + tn, 16)]
                    plv = pl_v[pl.ds(nb0 + tn, 16)]

                def scan(b, carry):
                    cum, fnd, bsel, cumb = carry
                    cb = plsc.load_gather(hist_v, [gb + b])
                    cum2 = cum + cb
                    hit = (~fnd) & (cum2 >= r)
                    bsel = jnp.where(hit, b, bsel)
                    cumb = jnp.where(hit, cum, cumb)
                    return cum2, fnd | hit, bsel, cumb

                cum, fnd, bsel, cumb = lax.fori_loop(
                    0, 1 << w, scan,
                    (zero16, jnp.full((16,), False), zero16, zero16))
                rnew = jnp.where(fnd, r - cumb, r)
                if s0 >= LO:
                    ph2 = jnp.where(fnd, ph | (bsel << (s0 - LO)), ph)
                    pl2 = plv
                elif s0 + w <= LO:
                    ph2 = ph
                    pl2 = jnp.where(fnd, plv | (bsel << s0), plv)
                else:
                    ph2 = jnp.where(fnd, ph | (bsel >> (LO - s0)), ph)
                    pl2 = jnp.where(
                        fnd, plv | ((bsel & ((1 << (LO - s0)) - 1)) << s0), plv)
                sl = pl.ds(tn, 16)
                pho_v[sl] = ph2
                plo_v[sl] = pl2
                ro_v[sl] = rnew
                return carry
            lax.fori_loop(0, NT // 32, ng, 0)

        osl = pl.ds(nb0, NT)
        pltpu.sync_copy(pho_v, ph_out.at[osl])
        pltpu.sync_copy(plo_v, pl_out.at[osl])
        pltpu.sync_copy(ro_v, r_out.at[osl])
        if first:
            pltpu.sync_copy(cnto_v, cnt_out.at[osl])

    i32v = jax.ShapeDtypeStruct((NPAD,), jnp.int32)
    out_type = [i32v, i32v, i32v, i32v] if first else [i32v, i32v, i32v]
    return functools.partial(
        pl.kernel,
        out_type=out_type,
        mesh=_mesh,
        compiler_params=_sc_params,
        scratch_types=[
            pltpu.VMEM((NPAD,), jnp.int32),
            pltpu.VMEM((NPAD,), jnp.int32),
            pltpu.VMEM((ECH,), jnp.int32),
            pltpu.VMEM((ECH,), jnp.int32),
            pltpu.VMEM((ROWS, 128), jnp.int32),
            pltpu.VMEM((ROWS, 128), jnp.int32),
            pltpu.VMEM((HHALF,), jnp.int32),
            pltpu.VMEM((NT,), jnp.int32),
            pltpu.VMEM((NT,), jnp.int32),
            pltpu.VMEM((NT,), jnp.int32),
            pltpu.VMEM((NT,), jnp.int32),
            pltpu.VMEM((NT,), jnp.int32),
            pltpu.VMEM_SHARED((NH * NB,), jnp.int32),
            pltpu.SemaphoreType.DMA,
        ],
    )(body)


_passes = [_make_pass(p) for p in range(len(WINDOWS))]


NHA = NH + 8


def _sc_agg_body(src_hbm, dst_hbm, bits_hbm, ph_hbm, plh_hbm, y_hbm, agg_out,
                 ph_v, pl_v, src_v, dst_v, bits_v, ssel_v, dsel_v,
                 rows_v, rows2_v, idx2d, zbuf, agg_sh, sem):
    c = lax.axis_index("c")
    t = lax.axis_index("s")
    iota = lax.iota(jnp.int32, 16)
    zf16 = jnp.zeros((16,), jnp.float32)

    def zb(i, carry):
        zbuf[i // 4, pl.ds((i % 4) * 16, 16)] = zf16
        return carry
    lax.fori_loop(0, 32 * 4, zb, 0)

    def za(i, carry):
        pltpu.sync_copy(zbuf, agg_sh.at[pl.ds(t * NT + i * 32, 32)])
        return carry
    lax.fori_loop(0, NT // 32, za, 0)

    @pl.when(t == 15)
    def _():
        pltpu.sync_copy(zbuf.at[pl.ds(0, 8)], agg_sh.at[pl.ds(NH, 8)])

    pltpu.sync_copy(ph_hbm, ph_v)
    pltpu.sync_copy(plh_hbm, pl_v)
    plsc.subcore_barrier()

    nsel = jnp.int32(0)
    for sub in range(NSUB):
        base = t * EC + sub * ECH
        cp1 = pltpu.async_copy(src_hbm.at[pl.ds(base, ECH)], src_v, sem)
        cp2 = pltpu.async_copy(dst_hbm.at[pl.ds(base, ECH)], dst_v, sem)
        cp3 = pltpu.async_copy(bits_hbm.at[pl.ds(base, ECH)], bits_v, sem)
        cp1.wait()
        cp2.wait()
        cp3.wait()

        def eb(k, ns):
            sl = pl.ds(k * 16, 16)
            d = dst_v[sl]
            sr = src_v[sl]
            bts = bits_v[sl]
            loidx = base + k * 16 + iota
            phd = plsc.load_gather(ph_v, [d])
            pld = plsc.load_gather(pl_v, [d])
            kept = (bts < phd) | ((bts == phd) & (loidx <= pld))
            m = kept & (d >= c * NH) & (d < (c + 1) * NH)
            plsc.store_compressed(ssel_v.at[pl.ds(ns, 16)], sr, mask=m)
            plsc.store_compressed(dsel_v.at[pl.ds(ns, 16)], d - c * NH, mask=m)
            return ns + jnp.sum(m.astype(jnp.int32))
        nsel = lax.fori_loop(0, VECS, eb, nsel)

    dummy_s = jnp.zeros((16,), jnp.int32)
    dummy_d = jnp.full((16,), NH, jnp.int32)

    def pad(i, carry):
        ssel_v[pl.ds(nsel + i * 16, 16)] = dummy_s
        dsel_v[pl.ds(nsel + i * 16, 16)] = dummy_d
        return carry
    lax.fori_loop(0, 16, pad, 0)

    nblk = (nsel + 255) // 256

    def blk(j, carry):
        g0 = pltpu.async_copy(
            y_hbm.at[ssel_v.at[pl.ds(j * 256, 128)]], rows_v, sem)
        g1 = pltpu.async_copy(
            y_hbm.at[ssel_v.at[pl.ds(j * 256 + 128, 128)]], rows2_v, sem)
        g0.wait()
        g1.wait()
        for u in range(8):
            idx2d[0, pl.ds(u * 16, 16)] = dsel_v[pl.ds(j * 256 + u * 16, 16)]
            idx2d[1, pl.ds(u * 16, 16)] = dsel_v[
                pl.ds(j * 256 + 128 + u * 16, 16)]
        s0 = pltpu.async_copy(rows_v, agg_sh.at[idx2d.at[0]], sem, add=True)
        s1 = pltpu.async_copy(rows2_v, agg_sh.at[idx2d.at[1]], sem, add=True)
        s0.wait()
        s1.wait()
        return carry
    lax.fori_loop(0, nblk, blk, 0)
    plsc.subcore_barrier()

    pltpu.sync_copy(agg_sh.at[pl.ds(t * NT, NT)],
                    agg_out.at[c, pl.ds(t * NT, NT)])


_sc_agg = functools.partial(
        pl.kernel,
        out_type=jax.ShapeDtypeStruct((2, NH, H), jnp.float32),
        mesh=_mesh,
        compiler_params=pltpu.CompilerParams(
            needs_layout_passes=False, use_tc_tiling_on_sc=False),
        scratch_types=[
            pltpu.VMEM((NPAD,), jnp.int32),
            pltpu.VMEM((NPAD,), jnp.int32),
            pltpu.VMEM((ECH,), jnp.int32),
            pltpu.VMEM((ECH,), jnp.int32),
            pltpu.VMEM((ECH,), jnp.int32),
            pltpu.VMEM((EC + 512,), jnp.int32),
            pltpu.VMEM((EC + 512,), jnp.int32),
            pltpu.VMEM((128, H), jnp.float32),
            pltpu.VMEM((128, H), jnp.float32),
            pltpu.VMEM((2, 128), jnp.int32),
            pltpu.VMEM((32, H), jnp.float32),
            pltpu.VMEM_SHARED((NHA, H), jnp.float32),
            pltpu.SemaphoreType.DMA,
        ],
    )(_sc_agg_body)


def _tc_post_body(z_ref, agg_ref, cnt_ref, wl_ref, bl_ref, out_ref):
    agg = jnp.concatenate([agg_ref[0], agg_ref[1]], axis=0)[:N]
    cnt = cnt_ref[...][:N].astype(jnp.float32)
    mean = agg / jnp.maximum(cnt, 1.0)
    h = jax.nn.relu(z_ref[...] + mean)
    out_ref[...] = jnp.dot(h, wl_ref[...], preferred_element_type=jnp.float32) + bl_ref[...]


def _tc_post(z, agg, cnt, W_lin, b_lin):
    return pl.pallas_call(
        _tc_post_body,
        out_shape=jax.ShapeDtypeStruct((N, C), jnp.float32),
    )(z, agg, cnt.reshape(NPAD, 1), W_lin, b_lin.reshape(1, C))


def kernel(x, edge_index, W_dist, b_dist, W_self, W_neigh, b_sage, W_lin, b_lin):
    src = edge_index[0]
    dst = edge_index[1]
    s, score, z, y = _tc_pre(x, W_dist, b_dist, W_self, W_neigh, b_sage)
    score1 = score.reshape(N)
    bits = _sc_bits(score1, src, dst)
    ph, plv, r, cnt = _passes[0](dst, bits)
    for p in range(1, len(WINDOWS)):
        ph, plv, r = _passes[p](dst, bits, ph, plv, r)
    agg = _sc_agg(src, dst, bits, ph, plv, y)
    logits = _tc_post(z, agg, cnt, W_lin, b_lin)
    return logits, s


# speedup vs baseline: 14.3960x; 1.2095x over previous
import functools
import jax
import jax.numpy as jnp
from jax import lax
from jax.experimental import pallas as pl
from jax.experimental.pallas import tpu as pltpu
from jax.experimental.pallas import tpu_sc as plsc

N = 10000
E = 320000
D = 128
H = 64
C = 2

NPAD = 10240
NH = 5120
NT = 320
EC = E // 16
NSUB = 5
ECH = EC // NSUB
VECS = ECH // 16
ROWS = 32
LO = 19
NB = 128
WINDOWS = [(44, 7), (37, 7), (30, 7), (23, 7), (16, 7), (9, 7), (2, 7), (0, 2)]
HHALF = (NT // 2) * NB

_mesh = plsc.VectorSubcoreMesh(core_axis_name="c", subcore_axis_name="s",
                               num_cores=2, num_subcores=16)
_sc_params = pltpu.CompilerParams(needs_layout_passes=False)


def _det_masks(p):
    mh = ml = 0
    for s0, w in WINDOWS[:p]:
        m = (1 << w) - 1
        if s0 >= LO:
            mh |= m << (s0 - LO)
        elif s0 + w <= LO:
            ml |= m << s0
        else:
            mh |= m >> (LO - s0)
            ml |= (m & ((1 << (LO - s0)) - 1)) << s0
    return mh, ml


def _bucket(s0, w, bts, loidx):
    m = (1 << w) - 1
    if s0 >= LO:
        return (bts >> (s0 - LO)) & m
    if s0 + w <= LO:
        return (loidx >> s0) & m
    return ((bts << (LO - s0)) | (loidx >> s0)) & m


def _tc_pre_body(x_ref, wd_ref, bd_ref, ws_ref, wn_ref, bs_ref,
                 s_ref, score_ref, z_ref, y_ref):
    x = x_ref[...]
    s = jnp.dot(x, wd_ref[...], preferred_element_type=jnp.float32) + bd_ref[...]
    s_ref[...] = s
    score_ref[...] = jnp.tanh(s[:, 0:1])
    z_ref[...] = jnp.dot(x, ws_ref[...], preferred_element_type=jnp.float32) + bs_ref[...]
    y_ref[...] = jnp.dot(x, wn_ref[...], preferred_element_type=jnp.float32)


def _tc_pre(x, W_dist, b_dist, W_self, W_neigh, b_sage):
    return pl.pallas_call(
        _tc_pre_body,
        out_shape=[
            jax.ShapeDtypeStruct((N, 2), jnp.float32),
            jax.ShapeDtypeStruct((N, 1), jnp.float32),
            jax.ShapeDtypeStruct((N, H), jnp.float32),
            jax.ShapeDtypeStruct((N, H), jnp.float32),
        ],
    )(x, W_dist, b_dist.reshape(1, 2), W_self, W_neigh, b_sage.reshape(1, H))


EB = E // 32


def _sc_bits_body(score_hbm, src_hbm, dst_hbm, bits_hbm,
                  score_v, src_v, dst_v, out_v):
    c = lax.axis_index("c")
    t = lax.axis_index("s")
    w = t * 2 + c
    base = w * EB
    pltpu.sync_copy(score_hbm, score_v)
    pltpu.sync_copy(src_hbm.at[pl.ds(base, EB)], src_v)
    pltpu.sync_copy(dst_hbm.at[pl.ds(base, EB)], dst_v)

    @plsc.parallel_loop(0, EB, 16, unroll=8)
    def _(i):
        sl = pl.ds(i, 16)
        a = plsc.load_gather(score_v, [src_v[sl]])
        b = plsc.load_gather(score_v, [dst_v[sl]])
        out_v[sl] = plsc.bitcast(jnp.abs(a - b), jnp.int32)
    pltpu.sync_copy(out_v, bits_hbm.at[pl.ds(base, EB)])


_sc_bits = functools.partial(
    pl.kernel,
    out_type=jax.ShapeDtypeStruct((E,), jnp.int32),
    mesh=_mesh,
    compiler_params=_sc_params,
    scratch_types=[
        pltpu.VMEM((N,), jnp.float32),
        pltpu.VMEM((EB,), jnp.int32),
        pltpu.VMEM((EB,), jnp.int32),
        pltpu.VMEM((EB,), jnp.int32),
    ],
)(_sc_bits_body)


def _i32c(v):
    return v - (1 << 32) if v >= (1 << 31) else v


def _make_pass(p):
    s0, w = WINDOWS[p]
    mh, ml = _det_masks(p)
    mh, ml = _i32c(mh), _i32c(ml)
    first = p == 0

    def body(*refs):
        zero16 = jnp.zeros((16,), jnp.int32)
        if first:
            (dst_hbm, bits_hbm,
             ph_out, pl_out, r_out, cnt_out,
             ph_v, pl_v, dst_v, bits_v, idx_st, val_st,
             hist_v, r_v, pho_v, plo_v, ro_v, cnto_v, hist_sh, sem) = refs
        else:
            (dst_hbm, bits_hbm, ph_hbm, plh_hbm, r_hbm,
             ph_out, pl_out, r_out,
             ph_v, pl_v, dst_v, bits_v, idx_st, val_st,
             hist_v, r_v, pho_v, plo_v, ro_v, cnto_v, hist_sh, sem) = refs
        c = lax.axis_index("c")
        t = lax.axis_index("s")
        iota = lax.iota(jnp.int32, 16)

        @plsc.parallel_loop(0, HHALF, 16, unroll=8)
        def _(i):
            hist_v[pl.ds(i, 16)] = zero16
        pltpu.sync_copy(hist_v, hist_sh.at[pl.ds(t * NT * NB, HHALF)])
        pltpu.sync_copy(hist_v, hist_sh.at[pl.ds(t * NT * NB + HHALF, HHALF)])
        if not first:
            pltpu.sync_copy(ph_hbm, ph_v)
            pltpu.sync_copy(plh_hbm, pl_v)
        plsc.subcore_barrier()

        for sub in range(NSUB):
            base = t * EC + sub * ECH
            cp1 = pltpu.async_copy(dst_hbm.at[pl.ds(base, ECH)], dst_v, sem)
            cp2 = pltpu.async_copy(bits_hbm.at[pl.ds(base, ECH)], bits_v, sem)
            cp1.wait()
            cp2.wait()

            for u in range(VECS % 8, 8):
                val_st[VECS // 8, pl.ds(u * 16, 16)] = zero16
                idx_st[VECS // 8, pl.ds(u * 16, 16)] = zero16
            for rr in range(VECS // 8 + 1, ROWS):
                for u in range(8):
                    val_st[rr, pl.ds(u * 16, 16)] = zero16
                    idx_st[rr, pl.ds(u * 16, 16)] = zero16

            @plsc.parallel_loop(0, ECH, 16, unroll=8)
            def _(e0):
                sl = pl.ds(e0, 16)
                d = dst_v[sl]
                bts = bits_v[sl]
                loidx = base + e0 + iota
                if first:
                    act = jnp.full((16,), True)
                else:
                    phd = plsc.load_gather(ph_v, [d])
                    pld = plsc.load_gather(pl_v, [d])
                    act = ((bts & mh) == phd) & ((loidx & ml) == pld)
                b = _bucket(s0, w, bts, loidx)
                inhalf = (d >= c * NH) & (d < (c + 1) * NH)
                val = (act & inhalf).astype(jnp.int32)
                dl = jnp.clip(d - c * NH, 0, NH - 1)
                slot = dl * NB + b
                row = e0 // 128
                col = e0 % 128
                idx_st[row, pl.ds(col, 16)] = slot
                val_st[row, pl.ds(col, 16)] = val

            def fire(j, carry):
                pltpu.async_copy(
                    val_st.at[j], hist_sh.at[idx_st.at[j]], sem, add=True)
                return carry
            lax.fori_loop(0, ROWS, fire, 0)

            def drain(j, carry):
                pltpu.make_async_copy(
                    val_st.at[j], hist_sh.at[idx_st.at[j]], sem).wait()
                return carry
            lax.fori_loop(0, ROWS, drain, 0)
        plsc.subcore_barrier()

        nb0 = c * NH + t * NT
        if not first:
            pltpu.sync_copy(r_hbm.at[pl.ds(nb0, NT)], r_v)

        for h in range(2):
            pltpu.sync_copy(
                hist_sh.at[pl.ds(t * NT * NB + h * HHALF, HHALF)], hist_v)

            def ng(g, carry, h=h):
                tn = h * (NT // 2) + g * 16
                gb = (g * 16 + iota) * NB
                if first:
                    def sb(b, cum):
                        return cum + plsc.load_gather(hist_v, [gb + b])
                    deg = plsc.parallel_loop(
                        0, 1 << w, 1, unroll=8, carry=zero16)(sb)
                    keep = jnp.maximum(deg >> 1, 1)
                    r = keep
                    cnto_v[pl.ds(tn, 16)] = jnp.where(deg > 0, keep, 0)
                    ph = zero16
                    plv = zero16
                else:
                    r = r_v[pl.ds(tn, 16)]
                    ph = ph_v[pl.ds(nb0 + tn, 16)]
                    plv = pl_v[pl.ds(nb0 + tn, 16)]

                def scan(b, carry):
                    cum, fnd, bsel, cumb = carry
                    cb = plsc.load_gather(hist_v, [gb + b])
                    cum2 = cum + cb
                    hit = (~fnd) & (cum2 >= r)
                    bsel = jnp.where(hit, b, bsel)
                    cumb = jnp.where(hit, cum, cumb)
                    return cum2, fnd | hit, bsel, cumb

                cum, fnd, bsel, cumb = plsc.parallel_loop(
                    0, 1 << w, 1, unroll=8,
                    carry=(zero16, jnp.full((16,), False), zero16, zero16))(scan)
                rnew = jnp.where(fnd, r - cumb, r)
                if s0 >= LO:
                    ph2 = jnp.where(fnd, ph | (bsel << (s0 - LO)), ph)
                    pl2 = plv
                elif s0 + w <= LO:
                    ph2 = ph
                    pl2 = jnp.where(fnd, plv | (bsel << s0), plv)
                else:
                    ph2 = jnp.where(fnd, ph | (bsel >> (LO - s0)), ph)
                    pl2 = jnp.where(
                        fnd, plv | ((bsel & ((1 << (LO - s0)) - 1)) << s0), plv)
                sl = pl.ds(tn, 16)
                pho_v[sl] = ph2
                plo_v[sl] = pl2
                ro_v[sl] = rnew
                return carry
            lax.fori_loop(0, NT // 32, ng, 0)

        osl = pl.ds(nb0, NT)
        pltpu.sync_copy(pho_v, ph_out.at[osl])
        pltpu.sync_copy(plo_v, pl_out.at[osl])
        pltpu.sync_copy(ro_v, r_out.at[osl])
        if first:
            pltpu.sync_copy(cnto_v, cnt_out.at[osl])

    i32v = jax.ShapeDtypeStruct((NPAD,), jnp.int32)
    out_type = [i32v, i32v, i32v, i32v] if first else [i32v, i32v, i32v]
    return functools.partial(
        pl.kernel,
        out_type=out_type,
        mesh=_mesh,
        compiler_params=_sc_params,
        scratch_types=[
            pltpu.VMEM((NPAD,), jnp.int32),
            pltpu.VMEM((NPAD,), jnp.int32),
            pltpu.VMEM((ECH,), jnp.int32),
            pltpu.VMEM((ECH,), jnp.int32),
            pltpu.VMEM((ROWS, 128), jnp.int32),
            pltpu.VMEM((ROWS, 128), jnp.int32),
            pltpu.VMEM((HHALF,), jnp.int32),
            pltpu.VMEM((NT,), jnp.int32),
            pltpu.VMEM((NT,), jnp.int32),
            pltpu.VMEM((NT,), jnp.int32),
            pltpu.VMEM((NT,), jnp.int32),
            pltpu.VMEM((NT,), jnp.int32),
            pltpu.VMEM_SHARED((NH * NB,), jnp.int32),
            pltpu.SemaphoreType.DMA,
        ],
    )(body)


_passes = [_make_pass(p) for p in range(len(WINDOWS))]


NHA = NH + 8


def _sc_agg_body(src_hbm, dst_hbm, bits_hbm, ph_hbm, plh_hbm, y_hbm, agg_out,
                 ph_v, pl_v, src_v, dst_v, bits_v, ssel_v, dsel_v,
                 rows_v, rows2_v, idx2d, zbuf, agg_sh, sem):
    c = lax.axis_index("c")
    t = lax.axis_index("s")
    iota = lax.iota(jnp.int32, 16)
    zf16 = jnp.zeros((16,), jnp.float32)

    @plsc.parallel_loop(0, 32 * 4, 1, unroll=8)
    def _(i):
        zbuf[i // 4, pl.ds((i % 4) * 16, 16)] = zf16

    def za(i, carry):
        pltpu.sync_copy(zbuf, agg_sh.at[pl.ds(t * NT + i * 32, 32)])
        return carry
    lax.fori_loop(0, NT // 32, za, 0)

    @pl.when(t == 15)
    def _():
        pltpu.sync_copy(zbuf.at[pl.ds(0, 8)], agg_sh.at[pl.ds(NH, 8)])

    pltpu.sync_copy(ph_hbm, ph_v)
    pltpu.sync_copy(plh_hbm, pl_v)
    plsc.subcore_barrier()

    nsel = jnp.int32(0)
    for sub in range(NSUB):
        base = t * EC + sub * ECH
        cp1 = pltpu.async_copy(src_hbm.at[pl.ds(base, ECH)], src_v, sem)
        cp2 = pltpu.async_copy(dst_hbm.at[pl.ds(base, ECH)], dst_v, sem)
        cp3 = pltpu.async_copy(bits_hbm.at[pl.ds(base, ECH)], bits_v, sem)
        cp1.wait()
        cp2.wait()
        cp3.wait()

        def eb(k, ns):
            sl = pl.ds(k * 16, 16)
            d = dst_v[sl]
            sr = src_v[sl]
            bts = bits_v[sl]
            loidx = base + k * 16 + iota
            phd = plsc.load_gather(ph_v, [d])
            pld = plsc.load_gather(pl_v, [d])
            kept = (bts < phd) | ((bts == phd) & (loidx <= pld))
            m = kept & (d >= c * NH) & (d < (c + 1) * NH)
            plsc.store_compressed(ssel_v.at[pl.ds(ns, 16)], sr, mask=m)
            plsc.store_compressed(dsel_v.at[pl.ds(ns, 16)], d - c * NH, mask=m)
            return ns + jnp.sum(m.astype(jnp.int32))
        nsel = lax.fori_loop(0, VECS, eb, nsel)

    dummy_s = jnp.zeros((16,), jnp.int32)
    dummy_d = jnp.full((16,), NH, jnp.int32)

    def pad(i, carry):
        ssel_v[pl.ds(nsel + i * 16, 16)] = dummy_s
        dsel_v[pl.ds(nsel + i * 16, 16)] = dummy_d
        return carry
    lax.fori_loop(0, 16, pad, 0)

    nblk = (nsel + 255) // 256

    def blk(j, carry):
        g0 = pltpu.async_copy(
            y_hbm.at[ssel_v.at[pl.ds(j * 256, 128)]], rows_v, sem)
        g1 = pltpu.async_copy(
            y_hbm.at[ssel_v.at[pl.ds(j * 256 + 128, 128)]], rows2_v, sem)
        g0.wait()
        g1.wait()
        for u in range(8):
            idx2d[0, pl.ds(u * 16, 16)] = dsel_v[pl.ds(j * 256 + u * 16, 16)]
            idx2d[1, pl.ds(u * 16, 16)] = dsel_v[
                pl.ds(j * 256 + 128 + u * 16, 16)]
        s0 = pltpu.async_copy(rows_v, agg_sh.at[idx2d.at[0]], sem, add=True)
        s1 = pltpu.async_copy(rows2_v, agg_sh.at[idx2d.at[1]], sem, add=True)
        s0.wait()
        s1.wait()
        return carry
    lax.fori_loop(0, nblk, blk, 0)
    plsc.subcore_barrier()

    pltpu.sync_copy(agg_sh.at[pl.ds(t * NT, NT)],
                    agg_out.at[c, pl.ds(t * NT, NT)])


_sc_agg = functools.partial(
        pl.kernel,
        out_type=jax.ShapeDtypeStruct((2, NH, H), jnp.float32),
        mesh=_mesh,
        compiler_params=pltpu.CompilerParams(
            needs_layout_passes=False, use_tc_tiling_on_sc=False),
        scratch_types=[
            pltpu.VMEM((NPAD,), jnp.int32),
            pltpu.VMEM((NPAD,), jnp.int32),
            pltpu.VMEM((ECH,), jnp.int32),
            pltpu.VMEM((ECH,), jnp.int32),
            pltpu.VMEM((ECH,), jnp.int32),
            pltpu.VMEM((EC + 512,), jnp.int32),
            pltpu.VMEM((EC + 512,), jnp.int32),
            pltpu.VMEM((128, H), jnp.float32),
            pltpu.VMEM((128, H), jnp.float32),
            pltpu.VMEM((2, 128), jnp.int32),
            pltpu.VMEM((32, H), jnp.float32),
            pltpu.VMEM_SHARED((NHA, H), jnp.float32),
            pltpu.SemaphoreType.DMA,
        ],
    )(_sc_agg_body)


def _tc_post_body(z_ref, agg_ref, cnt_ref, wl_ref, bl_ref, out_ref):
    agg = jnp.concatenate([agg_ref[0], agg_ref[1]], axis=0)[:N]
    cnt = cnt_ref[...][:N].astype(jnp.float32)
    mean = agg / jnp.maximum(cnt, 1.0)
    h = jax.nn.relu(z_ref[...] + mean)
    out_ref[...] = jnp.dot(h, wl_ref[...], preferred_element_type=jnp.float32) + bl_ref[...]


def _tc_post(z, agg, cnt, W_lin, b_lin):
    return pl.pallas_call(
        _tc_post_body,
        out_shape=jax.ShapeDtypeStruct((N, C), jnp.float32),
    )(z, agg, cnt.reshape(NPAD, 1), W_lin, b_lin.reshape(1, C))


def kernel(x, edge_index, W_dist, b_dist, W_self, W_neigh, b_sage, W_lin, b_lin):
    src = edge_index[0]
    dst = edge_index[1]
    s, score, z, y = _tc_pre(x, W_dist, b_dist, W_self, W_neigh, b_sage)
    score1 = score.reshape(N)
    bits = _sc_bits(score1, src, dst)
    ph, plv, r, cnt = _passes[0](dst, bits)
    for p in range(1, len(WINDOWS)):
        ph, plv, r = _passes[p](dst, bits, ph, plv, r)
    agg = _sc_agg(src, dst, bits, ph, plv, y)
    logits = _tc_post(z, agg, cnt, W_lin, b_lin)
    return logits, s
